# baseline (speedup 1.0000x reference)
import functools

import jax
import jax.numpy as jnp
from jax import lax
from jax.experimental import pallas as pl
from jax.experimental.pallas import tpu as pltpu
from jax.experimental.pallas import tpu_sc as plsc

N_NODES = 10000
N_EDGES = 320000
D_IN = 128
D_HID = 128
N_CLS = 40

NP = 10240
EP = 327680
NC, NS = 2, 16
K = 128
RT = NP // NS
D3 = 48
N_PAIR = (EP // K) // NS

_MESH = plsc.VectorSubcoreMesh(core_axis_name="c", subcore_axis_name="s")


def _zero_fill(buf, nrows, d):
    zv = jnp.zeros((16,), jnp.float32)

    def rowbody(j, carry):
        for k2 in range(d // 16):
            buf[j, pl.ds(k2 * 16, 16)] = zv
        return carry

    lax.fori_loop(0, nrows, rowbody, 0)


def _staged_writeback(acc_sh, out_ref, stage, s):
    for t in range(RT // K):
        pltpu.sync_copy(acc_sh.at[pl.ds(s * RT + t * K, K)], stage)
        pltpu.sync_copy(stage, out_ref.at[pl.ds(s * RT + t * K, K)])


def _make_deg_kernel():
    @functools.partial(
        pl.kernel,
        out_type=jax.ShapeDtypeStruct((NC, NP, 16), jnp.float32),
        mesh=_MESH,
        compiler_params=pltpu.CompilerParams(use_tc_tiling_on_sc=False),
        scratch_types=[
            pltpu.VMEM((K,), jnp.int32),
            pltpu.VMEM((K, 16), jnp.float32),
            pltpu.VMEM((K, 16), jnp.float32),
            pltpu.VMEM_SHARED((NP, 16), jnp.float32),
            pltpu.SemaphoreType.DMA,
        ],
    )
    def deg_kernel(dst_hbm, ones_hbm, out_hbm, dst_v, ones_v, zb, acc_sh, sem):
        c = lax.axis_index("c")
        s = lax.axis_index("s")
        wid = c * NS + s
        ew = EP // (NC * NS)
        _zero_fill(zb, K, 16)
        for t in range(RT // K):
            pltpu.sync_copy(zb, acc_sh.at[pl.ds(s * RT + t * K, K)])
        pltpu.sync_copy(ones_hbm, ones_v)
        plsc.subcore_barrier()
        base = wid * ew

        def body(j, carry):
            off = base + j * K
            pltpu.sync_copy(dst_hbm.at[pl.ds(off, K)], dst_v)
            pltpu.sync_copy(ones_v, acc_sh.at[dst_v], add=True)
            return carry

        lax.fori_loop(0, ew // K, body, 0)
        plsc.subcore_barrier()
        _staged_writeback(acc_sh, out_hbm.at[c], ones_v, s)

    return deg_kernel


def _make_prop_kernel(D):
    n = N_PAIR
    assert n % 4 == 0

    @functools.partial(
        pl.kernel,
        out_type=jax.ShapeDtypeStruct((NP, D), jnp.float32),
        mesh=_MESH,
        compiler_params=pltpu.CompilerParams(use_tc_tiling_on_sc=False),
        scratch_types=[
            pltpu.VMEM((2, K), jnp.int32),
            pltpu.VMEM((2, K), jnp.int32),
            pltpu.VMEM((2, K), jnp.int32),
            pltpu.VMEM((2, K), jnp.int32),
            pltpu.VMEM((K, D), jnp.float32),
            pltpu.VMEM((K, D), jnp.float32),
            pltpu.VMEM_SHARED((NP, D), jnp.float32),
            pltpu.SemaphoreType.DMA,
            pltpu.SemaphoreType.DMA,
            pltpu.SemaphoreType.DMA,
            pltpu.SemaphoreType.DMA,
            pltpu.SemaphoreType.DMA,
            pltpu.SemaphoreType.DMA,
        ],
    )
    def prop_kernel(g_hbm, ei_hbm, out_hbm,
                    ei0, ei1, ei2, ei3, rows0, rows1, acc_sh,
                    is0, is1, is2, is3, gs0, gs1):
        c = lax.axis_index("c")
        s = lax.axis_index("s")
        cbase = s * n

        @pl.when(c == 0)
        def _core0():
            _zero_fill(rows0, K, D)
            for t in range(RT // K):
                pltpu.sync_copy(rows0, acc_sh.at[pl.ds(s * RT + t * K, K)])
            plsc.subcore_barrier()
            eib = (ei0, ei1, ei2, ei3)
            isems = (is0, is1, is2, is3)
            rb = (rows0, rows1)
            gsems = (gs0, gs1)

            def idx_start(j, u):
                pltpu.async_copy(ei_hbm.at[cbase + j], eib[u], isems[u])

            def idx_wait(j, u):
                pltpu.make_async_copy(ei_hbm.at[cbase + j], eib[u], isems[u]).wait()

            def g_start(u, ur):
                pltpu.async_copy(g_hbm.at[eib[u].at[0]], rb[ur], gsems[ur])

            def g_wait(u, ur):
                pltpu.make_async_copy(g_hbm.at[eib[u].at[0]], rb[ur], gsems[ur]).wait()

            for u in range(4):
                idx_start(u, u)
            idx_wait(0, 0)
            g_start(0, 0)
            idx_wait(1, 1)
            g_start(1, 1)

            def step(j, u):
                ur = u % 2
                g_wait(u, ur)
                pltpu.sync_copy(rb[ur], acc_sh.at[eib[u].at[1]], add=True)

                @pl.when(j + 4 < n)
                def _():
                    idx_start(j + 4, u)

                @pl.when(j + 2 < n)
                def _():
                    idx_wait(j + 2, (u + 2) % 4)
                    g_start((u + 2) % 4, ur)

            def body(i, carry):
                for u in range(4):
                    step(4 * i + u, u)
                return carry

            lax.fori_loop(0, n // 4, body, 0)
            plsc.subcore_barrier()
            _staged_writeback(acc_sh, out_hbm, rows0, s)

    return prop_kernel


_BLK = 2048
_GRID = NP // _BLK


def _dinv_block(degp):
    deg = degp[0, :, 0] + degp[1, :, 0] + 1.0
    return lax.rsqrt(deg)


def _mat1_body(x_ref, w_ref, degp_ref, out_ref):
    dinv = _dinv_block(degp_ref[...])
    h = jnp.dot(x_ref[...], w_ref[...], preferred_element_type=jnp.float32)
    out_ref[...] = dinv[:, None] * h


def _ep_mat_body(sp_ref, g_ref, degp_ref, b_ref, w_ref, out_ref):
    dinv = _dinv_block(degp_ref[...])
    h = dinv[:, None] * (sp_ref[...] + g_ref[...]) + b_ref[...]
    h = jnp.maximum(h, 0.0)
    out_ref[...] = dinv[:, None] * jnp.dot(h, w_ref[...],
                                           preferred_element_type=jnp.float32)


def _final_body(sp_ref, g_ref, degp_ref, b_ref, out_ref):
    dinv = _dinv_block(degp_ref[...])
    z = dinv[:, None] * (sp_ref[...] + g_ref[...]) + b_ref[...]
    m = jnp.max(z, axis=1, keepdims=True)
    lse = jnp.log(jnp.sum(jnp.exp(z - m), axis=1, keepdims=True)) + m
    out_ref[...] = (z - lse)[:, :N_CLS]


def _degp_spec():
    return pl.BlockSpec((2, _BLK, 16), lambda i: (0, i, 0))


def _full_spec(shape):
    return pl.BlockSpec(shape, lambda i: tuple(0 for _ in shape))


def _tc_mat1(xpad, W1, degp):
    return pl.pallas_call(
        _mat1_body,
        grid=(_GRID,),
        in_specs=[
            pl.BlockSpec((_BLK, D_IN), lambda i: (i, 0)),
            _full_spec((D_IN, D_HID)),
            _degp_spec(),
        ],
        out_specs=pl.BlockSpec((_BLK, D_HID), lambda i: (i, 0)),
        out_shape=jax.ShapeDtypeStruct((NP, D_HID), jnp.float32),
    )(xpad, W1, degp)


def _tc_ep_mat(sp, g, degp, b, W, d_out):
    return pl.pallas_call(
        _ep_mat_body,
        grid=(_GRID,),
        in_specs=[
            pl.BlockSpec((_BLK, D_HID), lambda i: (i, 0)),
            pl.BlockSpec((_BLK, D_HID), lambda i: (i, 0)),
            _degp_spec(),
            _full_spec((1, D_HID)),
            _full_spec((D_HID, d_out)),
        ],
        out_specs=pl.BlockSpec((_BLK, d_out), lambda i: (i, 0)),
        out_shape=jax.ShapeDtypeStruct((NP, d_out), jnp.float32),
    )(sp, g, degp, b, W)


def _tc_final(sp, g, degp, b):
    return pl.pallas_call(
        _final_body,
        grid=(_GRID,),
        in_specs=[
            pl.BlockSpec((_BLK, D3), lambda i: (i, 0)),
            pl.BlockSpec((_BLK, D3), lambda i: (i, 0)),
            _degp_spec(),
            _full_spec((1, D3)),
        ],
        out_specs=pl.BlockSpec((_BLK, N_CLS), lambda i: (i, 0)),
        out_shape=jax.ShapeDtypeStruct((NP, N_CLS), jnp.float32),
    )(sp, g, degp, b)


def kernel(x, edge_index, W1, b1, W2, b2, W3, b3):
    src = edge_index[0].astype(jnp.int32)
    dst = edge_index[1].astype(jnp.int32)
    pad_e = jnp.full((EP - N_EDGES,), N_NODES, dtype=jnp.int32)
    src_p = jnp.concatenate([src, pad_e])
    dst_p = jnp.concatenate([dst, pad_e])
    ei = jnp.stack([src_p.reshape(-1, K), dst_p.reshape(-1, K)], axis=1)
    xpad = jnp.zeros((NP, D_IN), jnp.float32).at[:N_NODES].set(x)

    ones16 = jnp.ones((K, 16), jnp.float32)

    W3p = jnp.zeros((D_HID, D3), jnp.float32).at[:, :N_CLS].set(W3)
    b1r = b1.reshape(1, D_HID)
    b2r = b2.reshape(1, D_HID)
    b3r = jnp.full((1, D3), -1e30, jnp.float32).at[0, :N_CLS].set(b3)

    deg_k = _make_deg_kernel()
    prop128 = _make_prop_kernel(D_HID)
    prop48 = _make_prop_kernel(D3)

    degp = deg_k(dst_p, ones16)

    g1 = _tc_mat1(xpad, W1, degp)
    s1 = prop128(g1, ei)
    g2 = _tc_ep_mat(s1, g1, degp, b1r, W2, D_HID)
    s2 = prop128(g2, ei)
    g3 = _tc_ep_mat(s2, g2, degp, b2r, W3p, D3)
    s3 = prop48(g3, ei)
    out = _tc_final(s3, g3, degp, b3r)
    return out[:N_NODES]


# speedup vs baseline: 2.0286x; 2.0286x over previous
import functools

import jax
import jax.numpy as jnp
from jax import lax
from jax.experimental import pallas as pl
from jax.experimental.pallas import tpu as pltpu
from jax.experimental.pallas import tpu_sc as plsc

N_NODES = 10000
N_EDGES = 320000
D_IN = 128
D_HID = 128
N_CLS = 40

NP = 10240
EP = 327680
NC, NS = 2, 16
K = 128
RT = NP // NS
D3 = 48
N_PAIR = (EP // K) // NS

_MESH = plsc.VectorSubcoreMesh(core_axis_name="c", subcore_axis_name="s")


def _zero_fill(buf, nrows, d):
    zv = jnp.zeros((16,), jnp.float32)

    def rowbody(j, carry):
        for k2 in range(d // 16):
            buf[j, pl.ds(k2 * 16, 16)] = zv
        return carry

    lax.fori_loop(0, nrows, rowbody, 0)


def _staged_writeback(acc_sh, out_ref, stage, s):
    for t in range(RT // K):
        pltpu.sync_copy(acc_sh.at[pl.ds(s * RT + t * K, K)], stage)
        pltpu.sync_copy(stage, out_ref.at[pl.ds(s * RT + t * K, K)])


def _make_deg_kernel():
    @functools.partial(
        pl.kernel,
        out_type=jax.ShapeDtypeStruct((NC, NP, 16), jnp.float32),
        mesh=_MESH,
        compiler_params=pltpu.CompilerParams(use_tc_tiling_on_sc=False),
        scratch_types=[
            pltpu.VMEM((K,), jnp.int32),
            pltpu.VMEM((K, 16), jnp.float32),
            pltpu.VMEM((K, 16), jnp.float32),
            pltpu.VMEM_SHARED((NP, 16), jnp.float32),
            pltpu.SemaphoreType.DMA,
        ],
    )
    def deg_kernel(dst_hbm, ones_hbm, out_hbm, dst_v, ones_v, zb, acc_sh, sem):
        c = lax.axis_index("c")
        s = lax.axis_index("s")
        wid = c * NS + s
        ew = EP // (NC * NS)
        _zero_fill(zb, K, 16)
        for t in range(RT // K):
            pltpu.sync_copy(zb, acc_sh.at[pl.ds(s * RT + t * K, K)])
        pltpu.sync_copy(ones_hbm, ones_v)
        plsc.subcore_barrier()
        base = wid * ew

        def body(j, carry):
            off = base + j * K
            pltpu.sync_copy(dst_hbm.at[pl.ds(off, K)], dst_v)
            pltpu.sync_copy(ones_v, acc_sh.at[dst_v], add=True)
            return carry

        lax.fori_loop(0, ew // K, body, 0)
        plsc.subcore_barrier()
        _staged_writeback(acc_sh, out_hbm.at[c], ones_v, s)

    return deg_kernel


def _make_prop_kernel(D):
    n = N_PAIR
    assert n % 4 == 0

    @functools.partial(
        pl.kernel,
        out_type=jax.ShapeDtypeStruct((NP, D), jnp.float32),
        mesh=_MESH,
        compiler_params=pltpu.CompilerParams(use_tc_tiling_on_sc=False),
        scratch_types=[
            pltpu.VMEM((2, K), jnp.int32),
            pltpu.VMEM((2, K), jnp.int32),
            pltpu.VMEM((2, K), jnp.int32),
            pltpu.VMEM((2, K), jnp.int32),
            pltpu.VMEM((K, D), jnp.float32),
            pltpu.VMEM((K, D), jnp.float32),
            pltpu.VMEM_SHARED((NP, D), jnp.float32),
            pltpu.SemaphoreType.DMA,
            pltpu.SemaphoreType.DMA,
            pltpu.SemaphoreType.DMA,
            pltpu.SemaphoreType.DMA,
            pltpu.SemaphoreType.DMA,
            pltpu.SemaphoreType.DMA,
        ],
    )
    def prop_kernel(g_hbm, ei_hbm, out_hbm,
                    ei0, ei1, ei2, ei3, rows0, rows1, acc_sh,
                    is0, is1, is2, is3, gs0, gs1):
        c = lax.axis_index("c")
        s = lax.axis_index("s")
        cbase = s * n

        @pl.when(c == 0)
        def _core0():
            _zero_fill(rows0, K, D)
            for t in range(RT // K):
                pltpu.sync_copy(rows0, acc_sh.at[pl.ds(s * RT + t * K, K)])
            plsc.subcore_barrier()
            eib = (ei0, ei1, ei2, ei3)
            isems = (is0, is1, is2, is3)
            rb = (rows0, rows1)
            gsems = (gs0, gs1)

            def idx_start(j, u):
                pltpu.async_copy(ei_hbm.at[cbase + j], eib[u], isems[u])

            def idx_wait(j, u):
                pltpu.make_async_copy(ei_hbm.at[cbase + j], eib[u], isems[u]).wait()

            def g_start(u, ur):
                pltpu.async_copy(g_hbm.at[eib[u].at[0]], rb[ur], gsems[ur])

            def g_wait(u, ur):
                pltpu.make_async_copy(g_hbm.at[eib[u].at[0]], rb[ur], gsems[ur]).wait()

            for u in range(4):
                idx_start(u, u)
            idx_wait(0, 0)
            g_start(0, 0)
            idx_wait(1, 1)
            g_start(1, 1)

            def step(j, u):
                ur = u % 2
                g_wait(u, ur)
                pltpu.sync_copy(rb[ur], acc_sh.at[eib[u].at[1]], add=True)

                @pl.when(j + 4 < n)
                def _():
                    idx_start(j + 4, u)

                @pl.when(j + 2 < n)
                def _():
                    idx_wait(j + 2, (u + 2) % 4)
                    g_start((u + 2) % 4, ur)

            def body(i, carry):
                for u in range(4):
                    step(4 * i + u, u)
                return carry

            lax.fori_loop(0, n // 4, body, 0)
            plsc.subcore_barrier()
            _staged_writeback(acc_sh, out_hbm, rows0, s)

    return prop_kernel


_BLK = 2048
_GRID = NP // _BLK


def _dinv_block(degp):
    deg = degp[0, :, 0] + degp[1, :, 0] + 1.0
    return lax.rsqrt(deg)


def _mat1_body(x_ref, w_ref, degp_ref, out_ref):
    dinv = _dinv_block(degp_ref[...])
    h = jnp.dot(x_ref[...], w_ref[...], preferred_element_type=jnp.float32)
    out_ref[...] = dinv[:, None] * h


def _ep_mat_body(sp_ref, g_ref, degp_ref, b_ref, w_ref, out_ref):
    dinv = _dinv_block(degp_ref[...])
    h = dinv[:, None] * (sp_ref[...] + g_ref[...]) + b_ref[...]
    h = jnp.maximum(h, 0.0)
    out_ref[...] = dinv[:, None] * jnp.dot(h, w_ref[...],
                                           preferred_element_type=jnp.float32)


def _final_body(sp_ref, g_ref, degp_ref, b_ref, out_ref):
    dinv = _dinv_block(degp_ref[...])
    z = dinv[:, None] * (sp_ref[...] + g_ref[...]) + b_ref[...]
    m = jnp.max(z, axis=1, keepdims=True)
    lse = jnp.log(jnp.sum(jnp.exp(z - m), axis=1, keepdims=True)) + m
    out_ref[...] = (z - lse)[:, :N_CLS]


def _degp_spec():
    return pl.BlockSpec((2, _BLK, 16), lambda i: (0, i, 0))


def _full_spec(shape):
    return pl.BlockSpec(shape, lambda i: tuple(0 for _ in shape))


def _tc_mat1(xpad, W1, degp):
    return pl.pallas_call(
        _mat1_body,
        grid=(_GRID,),
        in_specs=[
            pl.BlockSpec((_BLK, D_IN), lambda i: (i, 0)),
            _full_spec((D_IN, D_HID)),
            _degp_spec(),
        ],
        out_specs=pl.BlockSpec((_BLK, D_HID), lambda i: (i, 0)),
        out_shape=jax.ShapeDtypeStruct((NP, D_HID), jnp.float32),
    )(xpad, W1, degp)


def _tc_ep_mat(sp, g, degp, b, W, d_out):
    return pl.pallas_call(
        _ep_mat_body,
        grid=(_GRID,),
        in_specs=[
            pl.BlockSpec((_BLK, D_HID), lambda i: (i, 0)),
            pl.BlockSpec((_BLK, D_HID), lambda i: (i, 0)),
            _degp_spec(),
            _full_spec((1, D_HID)),
            _full_spec((D_HID, d_out)),
        ],
        out_specs=pl.BlockSpec((_BLK, d_out), lambda i: (i, 0)),
        out_shape=jax.ShapeDtypeStruct((NP, d_out), jnp.float32),
    )(sp, g, degp, b, W)


def _tc_final(sp, g, degp, b):
    return pl.pallas_call(
        _final_body,
        grid=(_GRID,),
        in_specs=[
            pl.BlockSpec((_BLK, D3), lambda i: (i, 0)),
            pl.BlockSpec((_BLK, D3), lambda i: (i, 0)),
            _degp_spec(),
            _full_spec((1, D3)),
        ],
        out_specs=pl.BlockSpec((_BLK, N_CLS), lambda i: (i, 0)),
        out_shape=jax.ShapeDtypeStruct((NP, N_CLS), jnp.float32),
    )(sp, g, degp, b)


def kernel(x, edge_index, W1, b1, W2, b2, W3, b3):
    src = edge_index[0].astype(jnp.int32)
    dst = edge_index[1].astype(jnp.int32)
    pad_e = (N_NODES +
             jnp.arange(EP - N_EDGES, dtype=jnp.int32) % (NP - N_NODES))
    src_p = jnp.concatenate([src, pad_e])
    dst_p = jnp.concatenate([dst, pad_e])
    ei = jnp.stack([src_p.reshape(-1, K), dst_p.reshape(-1, K)], axis=1)
    xpad = jnp.zeros((NP, D_IN), jnp.float32).at[:N_NODES].set(x)

    ones16 = jnp.ones((K, 16), jnp.float32)

    W3p = jnp.zeros((D_HID, D3), jnp.float32).at[:, :N_CLS].set(W3)
    b1r = b1.reshape(1, D_HID)
    b2r = b2.reshape(1, D_HID)
    b3r = jnp.full((1, D3), -1e30, jnp.float32).at[0, :N_CLS].set(b3)

    deg_k = _make_deg_kernel()
    prop128 = _make_prop_kernel(D_HID)
    prop48 = _make_prop_kernel(D3)

    degp = deg_k(dst_p, ones16)

    g1 = _tc_mat1(xpad, W1, degp)
    s1 = prop128(g1, ei)
    g2 = _tc_ep_mat(s1, g1, degp, b1r, W2, D_HID)
    s2 = prop128(g2, ei)
    g3 = _tc_ep_mat(s2, g2, degp, b2r, W3p, D3)
    s3 = prop48(g3, ei)
    out = _tc_final(s3, g3, degp, b3r)
    return out[:N_NODES]
